# Initial kernel scaffold; baseline (speedup 1.0000x reference)
import jax
import jax.numpy as jnp
from jax.experimental import pallas as pl


def kernel(x, edge_index, edge_weight, edge_attr, atom_types, seq_neighs, lin1_w, fnet_w1, fnet_b1, fnet_w2, fnet_b2, lin2_w, lin2_b, lin_w, lin_b):
    raise NotImplementedError("write your pallas kernel here")


# speedup vs baseline: 1.5967x; 1.5967x over previous
import functools

import jax
import jax.numpy as jnp
import numpy as np
from jax import lax
from jax.experimental import pallas as pl
from jax.experimental.pallas import tpu as pltpu
from jax.experimental.pallas import tpu_sc as plsc

_N = 10000
_E = 320000
_H = 128
_RBF = 16
_CUTOFF = 5.0
_LOG2 = float(np.log(2.0))

_NC = 2
_NS = 16
_NW = _NC * _NS
_C = 128
_CHUNKS = _E // _C
_BASE_CH = _CHUNKS // _NW
_EXTRA = _CHUNKS % _NW
_RPT = 624
_TAIL = _N - _NS * _RPT


def _wf_body(ea_ref, ew_ref, w1_ref, b1_ref, w2_ref, b2_ref, o_ref):
    v = jnp.dot(ea_ref[...], w1_ref[...], preferred_element_type=jnp.float32)
    v = v + b1_ref[...]
    v = jnp.maximum(v, 0.0) + jnp.log(1.0 + jnp.exp(-jnp.abs(v))) - _LOG2
    v = jnp.dot(v, w2_ref[...], preferred_element_type=jnp.float32) + b2_ref[...]
    cut = 0.5 * (jnp.cos(ew_ref[...] * (np.pi / _CUTOFF)) + 1.0)
    o_ref[...] = v * cut


def _filters(edge_attr, edge_weight, fnet_w1, fnet_b1, fnet_w2, fnet_b2):
    be = 4000
    grid = (_E // be,)
    return pl.pallas_call(
        _wf_body,
        grid=grid,
        in_specs=[
            pl.BlockSpec((be, _RBF), lambda i: (i, 0)),
            pl.BlockSpec((be, 1), lambda i: (i, 0)),
            pl.BlockSpec((_RBF, _H), lambda i: (0, 0)),
            pl.BlockSpec((1, _H), lambda i: (0, 0)),
            pl.BlockSpec((_H, _H), lambda i: (0, 0)),
            pl.BlockSpec((1, _H), lambda i: (0, 0)),
        ],
        out_specs=pl.BlockSpec((be, _H), lambda i: (i, 0)),
        out_shape=jax.ShapeDtypeStruct((_E, _H), jnp.float32),
    )(edge_attr, edge_weight.reshape(_E, 1), fnet_w1,
      fnet_b1.reshape(1, _H), fnet_w2, fnet_b2.reshape(1, _H))


def _h_body(x_ref, w_ref, o_ref):
    o_ref[...] = jnp.dot(x_ref[...], w_ref[...],
                         preferred_element_type=jnp.float32)


def _node_transform(x, lin1_w):
    bn = 2000
    return pl.pallas_call(
        _h_body,
        grid=(_N // bn,),
        in_specs=[
            pl.BlockSpec((bn, _H), lambda i: (i, 0)),
            pl.BlockSpec((_H, _H), lambda i: (0, 0)),
        ],
        out_specs=pl.BlockSpec((bn, _H), lambda i: (i, 0)),
        out_shape=jax.ShapeDtypeStruct((_N, _H), jnp.float32),
    )(x, lin1_w)


def _sc_body(h_hbm, wf_hbm, src_hbm, dst_hbm, zero_hbm, out_hbm,
             src_v, dst_v, rows_v, wf_v, zbuf_v, agg_sh, gsem):
    cid = lax.axis_index("c")
    sid = lax.axis_index("s")
    wid = sid * _NC + cid

    pltpu.sync_copy(zero_hbm, zbuf_v)
    row0 = sid * _RPT
    done = 0
    while done < _RPT:
        ln = min(_C, _RPT - done)
        pltpu.sync_copy(zbuf_v.at[pl.ds(0, ln)],
                        agg_sh.at[pl.ds(row0 + done, ln)])
        done += ln

    @pl.when(sid == _NS - 1)
    def _():
        pltpu.sync_copy(zbuf_v.at[pl.ds(0, _TAIL)],
                        agg_sh.at[pl.ds(_NS * _RPT, _TAIL)])

    plsc.subcore_barrier()

    def process_chunk(chunk):
        base = chunk * _C
        pltpu.sync_copy(src_hbm.at[pl.ds(base, _C)], src_v)
        pltpu.sync_copy(dst_hbm.at[pl.ds(base, _C)], dst_v)
        pltpu.sync_copy(wf_hbm.at[pl.ds(base, _C)], wf_v)
        pltpu.async_copy(h_hbm.at[src_v], rows_v, gsem).wait()

        def edge_body(e, carry):
            for j in range(_H // 16):
                sl = pl.ds(j * 16, 16)
                rows_v[e, sl] = rows_v[e, sl] * wf_v[e, sl]
            return carry

        lax.fori_loop(0, _C, edge_body, 0)
        pltpu.sync_copy(rows_v, agg_sh.at[dst_v], add=True)

    def chunk_body(i, carry):
        process_chunk(wid + i * _NW)
        return carry

    lax.fori_loop(0, _BASE_CH, chunk_body, 0)

    @pl.when(wid < _EXTRA)
    def _():
        process_chunk(_BASE_CH * _NW + wid)

    plsc.subcore_barrier()
    pltpu.sync_copy(agg_sh.at[pl.ds(row0, _RPT)],
                    out_hbm.at[cid, pl.ds(row0, _RPT)])

    @pl.when(sid == _NS - 1)
    def _():
        pltpu.sync_copy(agg_sh.at[pl.ds(_NS * _RPT, _TAIL)],
                        out_hbm.at[cid, pl.ds(_NS * _RPT, _TAIL)])


def _aggregate(h, wf, src, dst):
    mesh = plsc.VectorSubcoreMesh(core_axis_name="c", subcore_axis_name="s")
    call = functools.partial(
        pl.kernel,
        out_type=jax.ShapeDtypeStruct((_NC, _N, _H), jnp.float32),
        mesh=mesh,
        scratch_types=[
            pltpu.VMEM((_C,), jnp.int32),
            pltpu.VMEM((_C,), jnp.int32),
            pltpu.VMEM((_C, _H), jnp.float32),
            pltpu.VMEM((_C, _H), jnp.float32),
            pltpu.VMEM((_C, _H), jnp.float32),
            pltpu.VMEM_SHARED((_N, _H), jnp.float32),
            pltpu.SemaphoreType.DMA,
        ],
    )(_sc_body)
    zero = jnp.zeros((_C, _H), jnp.float32)
    return call(h, wf, src, dst, zero)


def _out_body(a_ref, w2_ref, b2_ref, w3_ref, b3_ref, o_ref):
    a = a_ref[0] + a_ref[1]
    t = jnp.dot(a, w2_ref[...], preferred_element_type=jnp.float32)
    t = jnp.tanh(t + b2_ref[...])
    o_ref[...] = jnp.dot(t, w3_ref[...],
                         preferred_element_type=jnp.float32) + b3_ref[...]


def _tail(agg2, lin2_w, lin2_b, lin_w, lin_b):
    bn = 2000
    return pl.pallas_call(
        _out_body,
        grid=(_N // bn,),
        in_specs=[
            pl.BlockSpec((_NC, bn, _H), lambda i: (0, i, 0)),
            pl.BlockSpec((_H, _H), lambda i: (0, 0)),
            pl.BlockSpec((1, _H), lambda i: (0, 0)),
            pl.BlockSpec((_H, _H), lambda i: (0, 0)),
            pl.BlockSpec((1, _H), lambda i: (0, 0)),
        ],
        out_specs=pl.BlockSpec((bn, _H), lambda i: (i, 0)),
        out_shape=jax.ShapeDtypeStruct((_N, _H), jnp.float32),
    )(agg2, lin2_w, lin2_b.reshape(1, _H), lin_w, lin_b.reshape(1, _H))


def kernel(x, edge_index, edge_weight, edge_attr, atom_types, seq_neighs,
           lin1_w, fnet_w1, fnet_b1, fnet_w2, fnet_b2, lin2_w, lin2_b,
           lin_w, lin_b):
    src = edge_index[0]
    dst = edge_index[1]
    wf = _filters(edge_attr, edge_weight, fnet_w1, fnet_b1, fnet_w2, fnet_b2)
    h = _node_transform(x, lin1_w)
    agg2 = _aggregate(h, wf, src, dst)
    return _tail(agg2, lin2_w, lin2_b, lin_w, lin_b)


# speedup vs baseline: 2.8660x; 1.7950x over previous
import functools

import jax
import jax.numpy as jnp
import numpy as np
from jax import lax
from jax.experimental import pallas as pl
from jax.experimental.pallas import tpu as pltpu
from jax.experimental.pallas import tpu_sc as plsc

_N = 10000
_E = 320000
_H = 128
_RBF = 16
_CUTOFF = 5.0
_LOG2 = float(np.log(2.0))

_NC = 2
_NS = 16
_NW = _NC * _NS
_C = 128
_CHUNKS = _E // _C
_BASE_CH = _CHUNKS // _NW
_EXTRA = _CHUNKS % _NW
_RPT = 624
_TAIL = _N - _NS * _RPT


def _wf_body(ea_ref, ew_ref, w1_ref, b1_ref, w2_ref, b2_ref, o_ref, cut_ref):
    v = jnp.dot(ea_ref[...], w1_ref[...], preferred_element_type=jnp.float32)
    v = v + b1_ref[...]
    v = jnp.maximum(v, 0.0) + jnp.log(1.0 + jnp.exp(-jnp.abs(v))) - _LOG2
    v = jnp.dot(v, w2_ref[...], preferred_element_type=jnp.float32) + b2_ref[...]
    o_ref[...] = v

    @pl.when(pl.program_id(0) == 0)
    def _():
        cut_ref[...] = 0.5 * (jnp.cos(ew_ref[...] * (np.pi / _CUTOFF)) + 1.0)


def _filters(edge_attr, edge_weight, fnet_w1, fnet_b1, fnet_w2, fnet_b2):
    be = 6400
    grid = (_E // be,)
    return pl.pallas_call(
        _wf_body,
        grid=grid,
        in_specs=[
            pl.BlockSpec((be, _RBF), lambda i: (i, 0)),
            pl.BlockSpec((_E // 128, 128), lambda i: (0, 0)),
            pl.BlockSpec((_RBF, _H), lambda i: (0, 0)),
            pl.BlockSpec((1, _H), lambda i: (0, 0)),
            pl.BlockSpec((_H, _H), lambda i: (0, 0)),
            pl.BlockSpec((1, _H), lambda i: (0, 0)),
        ],
        out_specs=[
            pl.BlockSpec((be, _H), lambda i: (i, 0)),
            pl.BlockSpec((_E // 128, 128), lambda i: (0, 0)),
        ],
        out_shape=[
            jax.ShapeDtypeStruct((_E, _H), jnp.float32),
            jax.ShapeDtypeStruct((_E // 128, 128), jnp.float32),
        ],
    )(edge_attr, edge_weight.reshape(_E // 128, 128), fnet_w1,
      fnet_b1.reshape(1, _H), fnet_w2, fnet_b2.reshape(1, _H))


def _h_body(x_ref, w_ref, o_ref):
    o_ref[...] = jnp.dot(x_ref[...], w_ref[...],
                         preferred_element_type=jnp.float32)


def _node_transform(x, lin1_w):
    bn = 2000
    return pl.pallas_call(
        _h_body,
        grid=(_N // bn,),
        in_specs=[
            pl.BlockSpec((bn, _H), lambda i: (i, 0)),
            pl.BlockSpec((_H, _H), lambda i: (0, 0)),
        ],
        out_specs=pl.BlockSpec((bn, _H), lambda i: (i, 0)),
        out_shape=jax.ShapeDtypeStruct((_N, _H), jnp.float32),
    )(x, lin1_w)


def _sc_body(h_hbm, wf_hbm, cut_hbm, ei_hbm, zero_hbm, out_hbm,
             src_v, dst_v, cut_v, rows_v, wf_v, zbuf_v, agg_sh, gsem):
    cid = lax.axis_index("c")
    sid = lax.axis_index("s")
    wid = sid * _NC + cid

    pltpu.sync_copy(zero_hbm, zbuf_v)
    row0 = sid * _RPT
    done = 0
    while done < _RPT:
        ln = min(_C, _RPT - done)
        pltpu.sync_copy(zbuf_v.at[pl.ds(0, ln)],
                        agg_sh.at[pl.ds(row0 + done, ln)])
        done += ln

    @pl.when(sid == _NS - 1)
    def _():
        pltpu.sync_copy(zbuf_v.at[pl.ds(0, _TAIL)],
                        agg_sh.at[pl.ds(_NS * _RPT, _TAIL)])

    plsc.subcore_barrier()

    def process_chunk(chunk):
        base = chunk * _C
        pltpu.sync_copy(ei_hbm.at[0, pl.ds(base, _C)], src_v)
        pltpu.sync_copy(ei_hbm.at[1, pl.ds(base, _C)], dst_v)
        pltpu.sync_copy(cut_hbm.at[chunk], cut_v.at[pl.ds(0, _C)])
        pltpu.sync_copy(wf_hbm.at[pl.ds(base, _C)], wf_v)
        pltpu.async_copy(h_hbm.at[src_v], rows_v, gsem).wait()

        def edge_body(e, carry):
            s = cut_v[pl.ds(e, 16)][0]
            for j in range(_H // 16):
                sl = pl.ds(j * 16, 16)
                rows_v[e, sl] = rows_v[e, sl] * wf_v[e, sl] * s
            return carry

        lax.fori_loop(0, _C, edge_body, 0)
        pltpu.sync_copy(rows_v, agg_sh.at[dst_v], add=True)

    def chunk_body(i, carry):
        process_chunk(wid + i * _NW)
        return carry

    lax.fori_loop(0, _BASE_CH, chunk_body, 0)

    @pl.when(wid < _EXTRA)
    def _():
        process_chunk(_BASE_CH * _NW + wid)

    plsc.subcore_barrier()
    pltpu.sync_copy(agg_sh.at[pl.ds(row0, _RPT)],
                    out_hbm.at[cid, pl.ds(row0, _RPT)])

    @pl.when(sid == _NS - 1)
    def _():
        pltpu.sync_copy(agg_sh.at[pl.ds(_NS * _RPT, _TAIL)],
                        out_hbm.at[cid, pl.ds(_NS * _RPT, _TAIL)])


def _aggregate(h, wf, cut, edge_index):
    mesh = plsc.VectorSubcoreMesh(core_axis_name="c", subcore_axis_name="s")
    call = functools.partial(
        pl.kernel,
        out_type=jax.ShapeDtypeStruct((_NC, _N, _H), jnp.float32),
        mesh=mesh,
        scratch_types=[
            pltpu.VMEM((_C,), jnp.int32),
            pltpu.VMEM((_C,), jnp.int32),
            pltpu.VMEM((_C + 16,), jnp.float32),
            pltpu.VMEM((_C, _H), jnp.float32),
            pltpu.VMEM((_C, _H), jnp.float32),
            pltpu.VMEM((_C, _H), jnp.float32),
            pltpu.VMEM_SHARED((_N, _H), jnp.float32),
            pltpu.SemaphoreType.DMA,
        ],
    )(_sc_body)
    zero = jnp.zeros((_C, _H), jnp.float32)
    return call(h, wf, cut, edge_index, zero)


def _out_body(a_ref, w2_ref, b2_ref, w3_ref, b3_ref, o_ref):
    a = a_ref[0] + a_ref[1]
    t = jnp.dot(a, w2_ref[...], preferred_element_type=jnp.float32)
    t = jnp.tanh(t + b2_ref[...])
    o_ref[...] = jnp.dot(t, w3_ref[...],
                         preferred_element_type=jnp.float32) + b3_ref[...]


def _tail(agg2, lin2_w, lin2_b, lin_w, lin_b):
    bn = 2000
    return pl.pallas_call(
        _out_body,
        grid=(_N // bn,),
        in_specs=[
            pl.BlockSpec((_NC, bn, _H), lambda i: (0, i, 0)),
            pl.BlockSpec((_H, _H), lambda i: (0, 0)),
            pl.BlockSpec((1, _H), lambda i: (0, 0)),
            pl.BlockSpec((_H, _H), lambda i: (0, 0)),
            pl.BlockSpec((1, _H), lambda i: (0, 0)),
        ],
        out_specs=pl.BlockSpec((bn, _H), lambda i: (i, 0)),
        out_shape=jax.ShapeDtypeStruct((_N, _H), jnp.float32),
    )(agg2, lin2_w, lin2_b.reshape(1, _H), lin_w, lin_b.reshape(1, _H))


def kernel(x, edge_index, edge_weight, edge_attr, atom_types, seq_neighs,
           lin1_w, fnet_w1, fnet_b1, fnet_w2, fnet_b2, lin2_w, lin2_b,
           lin_w, lin_b):
    wf, cut = _filters(edge_attr, edge_weight, fnet_w1, fnet_b1,
                       fnet_w2, fnet_b2)
    h = _node_transform(x, lin1_w)
    agg2 = _aggregate(h, wf, cut, edge_index)
    return _tail(agg2, lin2_w, lin2_b, lin_w, lin_b)
